# Initial kernel scaffold; baseline (speedup 1.0000x reference)
import functools

import jax
import jax.numpy as jnp
from jax import lax
from jax.experimental import pallas as pl
from jax.experimental.pallas import tpu as pltpu
from jax.experimental.pallas import tpu_sc as plsc

B = 16384
F = 26
L = 20
V = 100000
D = 16
ND = 13
C = 16
KEEP_PROB = 0.8
TEMPERATURE = 0.1

NC = 2
NS = 16
NW = NC * NS
BAGS = B * F
PW = BAGS // NW
CB = 128
NCHUNK = PW // CB
RPC = CB * L
NIDX = RPC // 128


def _sc_embedding_bag(gid2d, tables_flat):
    mesh = plsc.VectorSubcoreMesh(core_axis_name="c", subcore_axis_name="s")

    @functools.partial(
        pl.kernel,
        mesh=mesh,
        out_type=jax.ShapeDtypeStruct((BAGS, D), jnp.float32),
        scratch_types=[
            pltpu.VMEM((NIDX, 128), jnp.int32),
            pltpu.VMEM((RPC, D), jnp.float32),
            pltpu.VMEM((CB, D), jnp.float32),
            pltpu.SemaphoreType.DMA,
        ],
    )
    def k(gid_hbm, tbl_hbm, out_hbm, idx_v, rows_v, out_v, sem):
        wid = lax.axis_index("s") * NC + lax.axis_index("c")
        base_bag = wid * PW

        def chunk_body(g, carry):
            bag0 = base_bag + g * CB
            row0 = (bag0 * L) // 128
            pltpu.sync_copy(gid_hbm.at[pl.ds(row0, NIDX)], idx_v)
            copies = []
            for j in range(NIDX):
                copies.append(pltpu.async_copy(
                    tbl_hbm.at[idx_v.at[j]],
                    rows_v.at[pl.ds(j * 128, 128)],
                    sem,
                ))
            for cp in copies:
                cp.wait()

            def pool_body(i, c2):
                e = i * L
                acc = rows_v[e]
                for l in range(1, L):
                    acc = acc + rows_v[e + l]
                out_v[i] = acc
                return c2
            lax.fori_loop(0, CB, pool_body, 0)
            pltpu.sync_copy(out_v, out_hbm.at[pl.ds(bag0, CB)])
            return carry

        lax.fori_loop(0, NCHUNK, chunk_body, 0)

    return k(gid2d, tables_flat)


BB = 2048


def _autodis_body(x_ref, w_ref, pm_ref, me_ref, ex_ref, o_ref):
    x = x_ref[...]
    xe = jnp.dot(x, ex_ref[...], preferred_element_type=jnp.float32)
    pre = xe * w_ref[...]
    h = jnp.where(pre >= 0, pre, 0.01 * pre)
    xb = jnp.dot(h, pm_ref[...], precision=lax.Precision.HIGHEST,
                 preferred_element_type=jnp.float32) + KEEP_PROB * h
    parts = []
    for n in range(ND):
        g = xb[:, n * C:(n + 1) * C] * (1.0 / TEMPERATURE)
        m = jnp.max(g, axis=1, keepdims=True)
        e = jnp.exp(g - m)
        parts.append(e / jnp.sum(e, axis=1, keepdims=True))
    xh = jnp.concatenate(parts, axis=1)
    o_ref[...] = jnp.dot(xh, me_ref[...], precision=lax.Precision.HIGHEST,
                         preferred_element_type=jnp.float32)


def _tc_autodis(dense_input, w_row, pm_bd, me_bd, expand):
    grid = (B // BB,)
    return pl.pallas_call(
        _autodis_body,
        grid=grid,
        in_specs=[
            pl.BlockSpec((BB, ND), lambda i: (i, 0)),
            pl.BlockSpec((1, ND * C), lambda i: (0, 0)),
            pl.BlockSpec((ND * C, ND * C), lambda i: (0, 0)),
            pl.BlockSpec((ND * C, ND * C), lambda i: (0, 0)),
            pl.BlockSpec((ND, ND * C), lambda i: (0, 0)),
        ],
        out_specs=pl.BlockSpec((BB, ND * C), lambda i: (i, 0)),
        out_shape=jax.ShapeDtypeStruct((B, ND * C), jnp.float32),
    )(dense_input, w_row, pm_bd, me_bd, expand)


def kernel(indices, dense_input, tables, meta_emb, proj_w, proj_m):
    idx32 = indices.astype(jnp.int32)
    offs = (jnp.arange(F, dtype=jnp.int32) * V)[None, :, None]
    gid2d = (idx32 + offs).reshape(BAGS * L // 128, 128)
    tables_flat = tables.reshape(F * V, D)

    eye = jnp.eye(ND, dtype=jnp.float32)
    pm_bd = jnp.einsum('mn,nij->mjni', eye, proj_m).reshape(ND * C, ND * C)
    me_bd = jnp.einsum('mn,ncd->mcnd', eye, meta_emb).reshape(ND * C, ND * C)
    w_row = proj_w.reshape(1, ND * C)
    expand = jnp.repeat(eye, C, axis=1)

    pooled = _sc_embedding_bag(gid2d, tables_flat)
    dense_out = _tc_autodis(dense_input, w_row, pm_bd, me_bd, expand)

    sparse_out = pooled.reshape(B, F * D)
    return jnp.concatenate([sparse_out, dense_out], axis=1)


# speedup vs baseline: 9.4781x; 9.4781x over previous
import functools

import jax
import jax.numpy as jnp
from jax import lax
from jax.experimental import pallas as pl
from jax.experimental.pallas import tpu as pltpu
from jax.experimental.pallas import tpu_sc as plsc

B = 16384
F = 26
L = 20
V = 100000
D = 16
ND = 13
C = 16
KEEP_PROB = 0.8
TEMPERATURE = 0.1

NC = 2
NS = 16
NW = NC * NS
BAGS = B * F
PW = BAGS // NW
CB = 128
NCHUNK = PW // CB
RPC = CB * L
NIDX = RPC // 128


def _sc_embedding_bag(gid2d, tables_flat):
    mesh = plsc.VectorSubcoreMesh(core_axis_name="c", subcore_axis_name="s")

    @functools.partial(
        pl.kernel,
        mesh=mesh,
        compiler_params=pltpu.CompilerParams(use_tc_tiling_on_sc=False),
        out_type=jax.ShapeDtypeStruct((BAGS, D), jnp.float32),
        scratch_types=[
            pltpu.VMEM((NIDX, 1, 128), jnp.int32),
            pltpu.VMEM((RPC, D), jnp.float32),
            pltpu.VMEM((CB, D), jnp.float32),
            pltpu.SemaphoreType.DMA,
        ],
    )
    def k(gid_hbm, tbl_hbm, out_hbm, idx_v, rows_v, out_v, sem):
        wid = lax.axis_index("s") * NC + lax.axis_index("c")
        base_bag = wid * PW

        def chunk_body(g, carry):
            bag0 = base_bag + g * CB
            row0 = (bag0 * L) // 128
            pltpu.sync_copy(gid_hbm.at[pl.ds(row0, NIDX)], idx_v)
            copies = []
            for j in range(NIDX):
                copies.append(pltpu.async_copy(
                    tbl_hbm.at[idx_v.at[j, 0]],
                    rows_v.at[pl.ds(j * 128, 128)],
                    sem,
                ))
            for cp in copies:
                cp.wait()

            def pool_body(i, c2):
                e = i * L
                acc = rows_v[e]
                for l in range(1, L):
                    acc = acc + rows_v[e + l]
                out_v[i] = acc
                return c2
            lax.fori_loop(0, CB, pool_body, 0)
            pltpu.sync_copy(out_v, out_hbm.at[pl.ds(bag0, CB)])
            return carry

        lax.fori_loop(0, NCHUNK, chunk_body, 0)

    return k(gid2d, tables_flat)


BB = 2048


def _autodis_body(x_ref, w_ref, pm_ref, me_ref, ex_ref, o_ref):
    x = x_ref[...]
    xe = jnp.dot(x, ex_ref[...], preferred_element_type=jnp.float32)
    pre = xe * w_ref[...]
    h = jnp.where(pre >= 0, pre, 0.01 * pre)
    xb = jnp.dot(h, pm_ref[...], precision=lax.Precision.HIGHEST,
                 preferred_element_type=jnp.float32) + KEEP_PROB * h
    parts = []
    for n in range(ND):
        g = xb[:, n * C:(n + 1) * C] * (1.0 / TEMPERATURE)
        m = jnp.max(g, axis=1, keepdims=True)
        e = jnp.exp(g - m)
        parts.append(e / jnp.sum(e, axis=1, keepdims=True))
    xh = jnp.concatenate(parts, axis=1)
    o_ref[...] = jnp.dot(xh, me_ref[...], precision=lax.Precision.HIGHEST,
                         preferred_element_type=jnp.float32)


def _tc_autodis(dense_input, w_row, pm_bd, me_bd, expand):
    grid = (B // BB,)
    return pl.pallas_call(
        _autodis_body,
        grid=grid,
        in_specs=[
            pl.BlockSpec((BB, ND), lambda i: (i, 0)),
            pl.BlockSpec((1, ND * C), lambda i: (0, 0)),
            pl.BlockSpec((ND * C, ND * C), lambda i: (0, 0)),
            pl.BlockSpec((ND * C, ND * C), lambda i: (0, 0)),
            pl.BlockSpec((ND, ND * C), lambda i: (0, 0)),
        ],
        out_specs=pl.BlockSpec((BB, ND * C), lambda i: (i, 0)),
        out_shape=jax.ShapeDtypeStruct((B, ND * C), jnp.float32),
    )(dense_input, w_row, pm_bd, me_bd, expand)


def kernel(indices, dense_input, tables, meta_emb, proj_w, proj_m):
    idx32 = indices.astype(jnp.int32)
    offs = (jnp.arange(F, dtype=jnp.int32) * V)[None, :, None]
    gid2d = (idx32 + offs).reshape(BAGS * L // 128, 1, 128)
    tables_flat = tables.reshape(F * V, D)

    eye = jnp.eye(ND, dtype=jnp.float32)
    pm_bd = jnp.einsum('mn,nij->mjni', eye, proj_m).reshape(ND * C, ND * C)
    me_bd = jnp.einsum('mn,ncd->mcnd', eye, meta_emb).reshape(ND * C, ND * C)
    w_row = proj_w.reshape(1, ND * C)
    expand = jnp.repeat(eye, C, axis=1)

    pooled = _sc_embedding_bag(gid2d, tables_flat)
    dense_out = _tc_autodis(dense_input, w_row, pm_bd, me_bd, expand)

    sparse_out = pooled.reshape(B, F * D)
    return jnp.concatenate([sparse_out, dense_out], axis=1)
